# Initial kernel scaffold; baseline (speedup 1.0000x reference)
import jax
import jax.numpy as jnp
from jax.experimental import pallas as pl


def kernel(x, weight):
    raise NotImplementedError("write your pallas kernel here")


# speedup vs baseline: 10.1929x; 10.1929x over previous
import functools

import jax
import jax.numpy as jnp
from jax import lax
from jax.experimental import pallas as pl
from jax.experimental.pallas import tpu as pltpu
from jax.experimental.pallas import tpu_sc as plsc

BUCKET = 100000
EMBED = 128
HIST = 200
BATCH = 4096

NUM_CORES = 2
NUM_SUBCORES = 16
LANES = 16
NW = NUM_CORES * NUM_SUBCORES
BPW = BATCH // NW
IDX_PER_W = BPW * HIST
VECS = EMBED // LANES
CHUNK0 = 128
CHUNK1 = HIST - CHUNK0


def _fire(w_hbm, idx_v, rows, sem, b):
    off = b * HIST
    pltpu.async_copy(w_hbm.at[idx_v.at[pl.ds(off, CHUNK0)]],
                     rows.at[pl.ds(0, CHUNK0)], sem)
    pltpu.async_copy(w_hbm.at[idx_v.at[pl.ds(off + CHUNK0, CHUNK1)]],
                     rows.at[pl.ds(CHUNK0, CHUNK1)], sem)


def _drain(w_hbm, rows, sem):
    pltpu.make_async_copy(w_hbm.at[pl.ds(0, HIST)], rows, sem).wait()


def _accum_row(rows, b, out_v):
    def step(j, acc):
        return tuple(acc[k] + rows[j, pl.ds(k * LANES, LANES)]
                     for k in range(VECS))

    zero = jnp.zeros((LANES,), jnp.float32)
    acc = lax.fori_loop(0, HIST, step, (zero,) * VECS)
    for k in range(VECS):
        out_v[b, pl.ds(k * LANES, LANES)] = acc[k]


def _body(x_hbm, w_hbm, out_hbm, idx_v, rows0, rows1, out_v, sem0, sem1):
    wid = lax.axis_index("s") * NUM_CORES + lax.axis_index("c")
    base = wid * BPW

    pltpu.sync_copy(x_hbm.at[pl.ds(wid * IDX_PER_W, IDX_PER_W)], idx_v)

    def hash_step(i, carry):
        v = idx_v[pl.ds(i * LANES, LANES)]
        idx_v[pl.ds(i * LANES, LANES)] = lax.rem(v, BUCKET)
        return carry

    lax.fori_loop(0, IDX_PER_W // LANES, hash_step, 0)

    _fire(w_hbm, idx_v, rows0, sem0, 0)

    @pl.loop(0, BPW, step=2)
    def row_loop(b):
        _fire(w_hbm, idx_v, rows1, sem1, b + 1)
        _drain(w_hbm, rows0, sem0)
        _accum_row(rows0, b, out_v)

        @pl.when(b + 2 < BPW)
        def _():
            _fire(w_hbm, idx_v, rows0, sem0, b + 2)

        _drain(w_hbm, rows1, sem1)
        _accum_row(rows1, b + 1, out_v)

    pltpu.sync_copy(out_v, out_hbm.at[pl.ds(base, BPW)])


@jax.jit
def kernel(x, weight):
    mesh = plsc.VectorSubcoreMesh(core_axis_name="c", subcore_axis_name="s",
                                  num_cores=NUM_CORES,
                                  num_subcores=NUM_SUBCORES)
    run = pl.kernel(
        _body,
        out_type=jax.ShapeDtypeStruct((BATCH, EMBED), jnp.float32),
        mesh=mesh,
        scratch_types=[
            pltpu.VMEM((IDX_PER_W,), jnp.int32),
            pltpu.VMEM((HIST, EMBED), jnp.float32),
            pltpu.VMEM((HIST, EMBED), jnp.float32),
            pltpu.VMEM((BPW, EMBED), jnp.float32),
            pltpu.SemaphoreType.DMA,
            pltpu.SemaphoreType.DMA,
        ],
    )
    return run(x.reshape(BATCH * HIST), weight)
